# baseline (speedup 1.0000x reference)
import functools

import jax
import jax.numpy as jnp
from jax import lax
from jax.experimental import pallas as pl
from jax.experimental.pallas import tpu as pltpu
from jax.experimental.pallas import tpu_sc as plsc

VOCAB = 100000
EMBED = 64
BATCH = 4096
TP = 8

_info = plsc.get_sparse_core_info()
_NC, _NS, _L = _info.num_cores, _info.num_subcores, _info.num_lanes
_NW = _NC * _NS
_CHUNKS = BATCH // TP
_DPW = EMBED // _NW
_CK = 128


def _sc_body(xp_hbm, tt_hbm, out_hbm, xbuf, rowbuf, sem):
    wid = lax.axis_index("s") * _NC + lax.axis_index("c")
    pltpu.sync_copy(xp_hbm, xbuf)
    pltpu.async_copy(tt_hbm.at[wid].at[xbuf], rowbuf, sem).wait()
    pltpu.sync_copy(rowbuf, out_hbm.at[wid])


_gather_half = functools.partial(
    pl.kernel,
    out_type=jax.ShapeDtypeStruct((_NW, BATCH), jnp.float32),
    mesh=plsc.VectorSubcoreMesh(core_axis_name="c", subcore_axis_name="s"),
    scratch_types=[
        pltpu.VMEM((BATCH,), jnp.int32),
        pltpu.VMEM((BATCH,), jnp.float32),
        pltpu.SemaphoreType.DMA,
    ],
    compiler_params=pltpu.CompilerParams(use_tc_tiling_on_sc=False),
)(_sc_body)


@jax.jit
def kernel(x, table):
    xp = jnp.asarray(x, jnp.int32).reshape(_CHUNKS, TP).T.reshape(BATCH)
    tt = table.T
    halves = [_gather_half(xp, tt[i * _NW:(i + 1) * _NW])
              for i in range(EMBED // _NW)]
    out_d = jnp.concatenate(halves, axis=0)
    return (out_d.reshape(EMBED, TP, _CHUNKS)
            .transpose(1, 2, 0).reshape(TP, _CHUNKS * EMBED))


# speedup vs baseline: 1.1795x; 1.1795x over previous
import functools

import jax
import jax.numpy as jnp
from jax import lax
from jax.experimental import pallas as pl
from jax.experimental.pallas import tpu as pltpu
from jax.experimental.pallas import tpu_sc as plsc

VOCAB = 100000
EMBED = 64
BATCH = 4096
TP = 8

_info = plsc.get_sparse_core_info()
_NC, _NS, _L = _info.num_cores, _info.num_subcores, _info.num_lanes
_NW = _NC * _NS
_CHUNKS = BATCH // TP
_DPW = EMBED // _NW
_CK = 128


def _sc_body(xp_hbm, tt_hbm, out_hbm, xbuf, rowbuf, sem):
    wid = lax.axis_index("s") * _NC + lax.axis_index("c")
    pltpu.sync_copy(xp_hbm, xbuf)
    copies = []
    for dd in range(_DPW):
        d = wid * _DPW + dd
        copies.append(pltpu.async_copy(
            tt_hbm.at[d].at[xbuf],
            rowbuf.at[pl.ds(dd * BATCH, BATCH)], sem))
    for c in copies:
        c.wait()
    pltpu.sync_copy(rowbuf, out_hbm.at[wid])


_gather_embed = functools.partial(
    pl.kernel,
    out_type=jax.ShapeDtypeStruct((_NW, _DPW * BATCH), jnp.float32),
    mesh=plsc.VectorSubcoreMesh(core_axis_name="c", subcore_axis_name="s"),
    scratch_types=[
        pltpu.VMEM((BATCH,), jnp.int32),
        pltpu.VMEM((_DPW * BATCH,), jnp.float32),
        pltpu.SemaphoreType.DMA,
    ],
    compiler_params=pltpu.CompilerParams(use_tc_tiling_on_sc=False),
)(_sc_body)


@jax.jit
def kernel(x, table):
    xp = jnp.asarray(x, jnp.int32).reshape(_CHUNKS, TP).T.reshape(BATCH)
    out_d = _gather_embed(xp, table.T)
    return (out_d.reshape(EMBED, TP, _CHUNKS)
            .transpose(1, 2, 0).reshape(TP, _CHUNKS * EMBED))
